# baseline (speedup 1.0000x reference)
import jax
import jax.numpy as jnp
from jax import lax
from jax.experimental import pallas as pl
from jax.experimental.pallas import tpu as pltpu
from jax.experimental.pallas import tpu_sc as plsc

B = 64
C = 768
D = 24 * 24

NB_SC = 16
NB_TC = B - NB_SC

NC = 2
NS = 16
NW = NC * NS
SC_ROWS = NB_SC * C
ROWS_PER_W = SC_ROWS // NW
CHUNK = 32
NCHUNK = ROWS_PER_W // CHUNK
NBUF = 4
LEAD = 2
HALVES = NW // NB_SC
JSPAN = C // HALVES


def _sc_shuffle(x_hbm, idx_hbm, out_hbm, idx_v, gidx_v,
                buf0, buf1, buf2, buf3,
                sem_g0, sem_g1, sem_g2, sem_g3,
                sem_o0, sem_o1, sem_o2, sem_o3):
    cid = lax.axis_index("c")
    sid = lax.axis_index("s")
    wid = sid * NC + cid
    batch = wid % NB_SC
    half = wid // NB_SC
    j_base = half * JSPAN
    row_base = batch * C
    out_base = row_base + j_base

    pltpu.sync_copy(idx_hbm, idx_v)

    for ci in range(NCHUNK):
        for ki in range(CHUNK // 16):
            off = j_base + ci * CHUNK + 16 * ki
            gidx_v[ci, pl.ds(16 * ki, 16)] = idx_v[pl.ds(off, 16)] + row_base

    bufs = (buf0, buf1, buf2, buf3)
    gsems = (sem_g0, sem_g1, sem_g2, sem_g3)
    osems = (sem_o0, sem_o1, sem_o2, sem_o3)
    gathers = [None] * NBUF
    outs = [None] * NBUF

    for ci in range(LEAD):
        gathers[ci % NBUF] = pltpu.async_copy(
            x_hbm.at[gidx_v.at[ci]], bufs[ci % NBUF], gsems[ci % NBUF])

    for ci in range(NCHUNK):
        b = ci % NBUF
        gathers[b].wait()
        outs[b] = pltpu.async_copy(
            bufs[b], out_hbm.at[pl.ds(out_base + ci * CHUNK, CHUNK)],
            osems[b])
        nxt = ci + LEAD
        if nxt < NCHUNK:
            bn = nxt % NBUF
            if outs[bn] is not None:
                outs[bn].wait()
                outs[bn] = None
            gathers[bn] = pltpu.async_copy(
                x_hbm.at[gidx_v.at[nxt]], bufs[bn], gsems[bn])

    for b in range(NBUF):
        if outs[b] is not None:
            outs[b].wait()


def _sc_part(x, forward_shuffle_idx):
    xr = x.reshape(B * C, D)
    mesh = plsc.VectorSubcoreMesh(core_axis_name="c", subcore_axis_name="s")
    run = pl.kernel(
        _sc_shuffle,
        out_type=jax.ShapeDtypeStruct((SC_ROWS, D), jnp.float32),
        mesh=mesh,
        scratch_types=[
            pltpu.VMEM((C,), jnp.int32),
            pltpu.VMEM((NCHUNK, CHUNK), jnp.int32),
            pltpu.VMEM((CHUNK, D), jnp.float32),
            pltpu.VMEM((CHUNK, D), jnp.float32),
            pltpu.VMEM((CHUNK, D), jnp.float32),
            pltpu.VMEM((CHUNK, D), jnp.float32),
            pltpu.SemaphoreType.DMA,
            pltpu.SemaphoreType.DMA,
            pltpu.SemaphoreType.DMA,
            pltpu.SemaphoreType.DMA,
            pltpu.SemaphoreType.DMA,
            pltpu.SemaphoreType.DMA,
            pltpu.SemaphoreType.DMA,
            pltpu.SemaphoreType.DMA,
        ],
        compiler_params=pltpu.CompilerParams(use_tc_tiling_on_sc=False),
    )
    return run(xr, forward_shuffle_idx).reshape(NB_SC, C, D)


BB = 4


def _tc_body(idx_ref, in_ref, out_ref):
    def body(j, carry):
        src = idx_ref[j]
        out_ref[:, pl.ds(j, 1), :] = in_ref[:, pl.ds(src, 1), :]
        return carry

    lax.fori_loop(0, C, body, 0, unroll=8)


def _tc_part(x, forward_shuffle_idx):
    xr = x.reshape(B, C, D)
    grid_spec = pltpu.PrefetchScalarGridSpec(
        num_scalar_prefetch=1,
        grid=(NB_TC // BB,),
        in_specs=[
            pl.BlockSpec((BB, C, D),
                         lambda i, idx_ref: (i + NB_SC // BB, 0, 0)),
        ],
        out_specs=pl.BlockSpec((BB, C, D), lambda i, idx_ref: (i, 0, 0)),
    )
    return pl.pallas_call(
        _tc_body,
        grid_spec=grid_spec,
        out_shape=jax.ShapeDtypeStruct((NB_TC, C, D), jnp.float32),
    )(forward_shuffle_idx, xr)


@jax.jit
def _shuffle(x, forward_shuffle_idx):
    sc_out = _sc_part(x, forward_shuffle_idx)
    tc_out = _tc_part(x, forward_shuffle_idx)
    out = jnp.concatenate([sc_out, tc_out], axis=0)
    return out.reshape(B, C, 24, 24)


def kernel(x, forward_shuffle_idx):
    return (_shuffle(x, forward_shuffle_idx), 0)
